# baseline (speedup 1.0000x reference)
import jax
import jax.numpy as jnp
from jax.experimental import pallas as pl
from jax.experimental.pallas import tpu as pltpu

_N, _P, _D = 4, 4096, 3
_QC = 1024
_NQ = _P // _QC


def _chamfer_kernel(src_ref, tgt_ref, out_ref):
    b = pl.program_id(0)

    S = src_ref[0]
    T = tgt_ref[0]

    S8 = jnp.concatenate(
        [S, jnp.zeros((_P, 8 - _D), jnp.float32)], axis=1)
    St8 = jnp.swapaxes(S8, 0, 1)
    St = St8[:_D]

    x2 = jnp.sum(St * St, axis=0, keepdims=True)
    y2 = jnp.sum(T * T, axis=1, keepdims=True)

    y2_hi = y2.astype(jnp.bfloat16).astype(jnp.float32)
    y2_lo = y2 - y2_hi
    L = jnp.concatenate([T, y2_hi, y2_lo], axis=1)
    ones_p = jnp.ones((1, _P), jnp.float32)
    R = jnp.concatenate([-2.0 * St, ones_p, ones_p],
                        axis=0)

    m = None
    for j in range(_NQ):
        d = jax.lax.dot_general(
            L[j * _QC:(j + 1) * _QC], R, (((1,), (0,)), ((), ())),
            preferred_element_type=jnp.float32,
        )
        mj = jnp.min(d, axis=0, keepdims=True)
        m = mj if m is None else jnp.minimum(m, mj)

    s = jnp.sum(m + x2, keepdims=True) * (1.0 / _N)

    @pl.when(b == 0)
    def _():
        out_ref[...] = jnp.zeros_like(out_ref)

    out_ref[...] += s


def kernel(source_cloud, target_cloud):
    out = pl.pallas_call(
        _chamfer_kernel,
        grid=(_N,),
        in_specs=[
            pl.BlockSpec((1, _P, _D), lambda b: (b, 0, 0)),
            pl.BlockSpec((1, _P, _D), lambda b: (b, 0, 0)),
        ],
        out_specs=pl.BlockSpec((1, 1), lambda b: (0, 0)),
        out_shape=jax.ShapeDtypeStruct((1, 1), jnp.float32),
    )(source_cloud, target_cloud)
    return out[0, 0]
